# Initial kernel scaffold; baseline (speedup 1.0000x reference)
import jax
import jax.numpy as jnp
from jax.experimental import pallas as pl


def kernel(x, edge_index, W0, b0, W1, b1, Wout, bout):
    raise NotImplementedError("write your pallas kernel here")


# speedup vs baseline: 23.4319x; 23.4319x over previous
import functools

import jax
import jax.numpy as jnp
from jax import lax
from jax.experimental import pallas as pl
from jax.experimental.pallas import tpu as pltpu
from jax.experimental.pallas import tpu_sc as plsc

N = 10000
E = 320000
D = 128
H = 64

NC = 2
NS = 16
NW = NC * NS
EPT = E // NW
K = 80
NCH = EPT // K
RPT = N // NS

BM = 1000


def _sc_mesh():
    return plsc.VectorSubcoreMesh(
        core_axis_name="c", subcore_axis_name="s", num_cores=NC, num_subcores=NS
    )


@functools.cache
def _deg_kernel_build():
    return pl.kernel(
        _deg_body,
        out_type=jax.ShapeDtypeStruct((NC * N,), jnp.float32),
        mesh=_sc_mesh(),
        scratch_types=[
            pltpu.VMEM((NCH, K), jnp.int32),
            pltpu.VMEM((K,), jnp.float32),
            pltpu.VMEM((640,), jnp.float32),
            pltpu.VMEM_SHARED((N,), jnp.float32),
        ],
    )


def _deg_body(dst_hbm, out_hbm, didx, ones, zbuf, acc):
    c = lax.axis_index("c")
    s = lax.axis_index("s")
    wid = c * NS + s
    row0 = s * 624
    for j in range(640 // 16):
        zbuf[pl.ds(j * 16, 16)] = jnp.zeros((16,), jnp.float32)
    for j in range(K // 16):
        ones[pl.ds(j * 16, 16)] = jnp.ones((16,), jnp.float32)
    pltpu.sync_copy(zbuf, acc.at[pl.ds(row0, 640)])
    pltpu.sync_copy(dst_hbm.at[wid], didx)
    plsc.subcore_barrier()

    def body(i, carry):
        pltpu.sync_copy(ones, acc.at[didx.at[i]], add=True)
        return carry

    lax.fori_loop(0, NCH, body, 0)
    plsc.subcore_barrier()
    pltpu.sync_copy(acc.at[pl.ds(row0, 640)], zbuf)
    pltpu.sync_copy(zbuf, out_hbm.at[pl.ds(c * N + row0, 640)])


@functools.cache
def _agg_kernel_build():
    return pl.kernel(
        _agg_body,
        out_type=jax.ShapeDtypeStruct((NC, N, H), jnp.float32),
        mesh=_sc_mesh(),
        scratch_types=[
            pltpu.VMEM((NCH, K), jnp.int32),
            pltpu.VMEM((NCH, K), jnp.int32),
            pltpu.VMEM((K, H), jnp.float32),
            pltpu.VMEM_SHARED((N, H), jnp.float32),
            pltpu.SemaphoreType.DMA,
        ],
        compiler_params=pltpu.CompilerParams(use_tc_tiling_on_sc=False),
    )


def _agg_body(src_hbm, dst_hbm, msc_hbm, out_hbm,
              sidx, didx, rows, acc, sem):
    c = lax.axis_index("c")
    s = lax.axis_index("s")
    wid = c * NS + s
    row0 = s * 624

    for j in range(640 // K):
        pltpu.sync_copy(msc_hbm.at[pl.ds(row0 + j * K, K)], rows)
        pltpu.sync_copy(rows, acc.at[pl.ds(row0 + j * K, K)])

    pltpu.sync_copy(src_hbm.at[wid], sidx)
    pltpu.sync_copy(dst_hbm.at[wid], didx)
    plsc.subcore_barrier()

    def body(i, carry):
        pltpu.async_copy(msc_hbm.at[sidx.at[i]], rows, sem).wait()
        pltpu.sync_copy(rows, acc.at[didx.at[i]], add=True)
        return carry

    lax.fori_loop(0, NCH, body, 0)
    plsc.subcore_barrier()
    for j in range(640 // K):
        pltpu.sync_copy(acc.at[pl.ds(row0 + j * K, K)], rows)
        pltpu.sync_copy(rows, out_hbm.at[c, pl.ds(row0 + j * K, K)])


def _dinv_of(degt_ref):
    d = degt_ref[...]
    return lax.rsqrt(1.0 + d[:, :1] + d[:, 1:2])


def _scale_mm_body(degt_ref, x_ref, w_ref, out_ref):
    dinv = _dinv_of(degt_ref)
    m = jnp.dot(x_ref[...], w_ref[...], preferred_element_type=jnp.float32)
    out_ref[...] = m * dinv


def _mid_body(degt_ref, aggp_ref, msc0_ref, b0_ref, w1_ref, h1_ref, msc1_ref):
    dinv = _dinv_of(degt_ref)
    agg = aggp_ref[0] + aggp_ref[1] - msc0_ref[...]
    h1 = jnp.maximum(agg * dinv + b0_ref[...], 0.0)
    h1_ref[...] = h1
    msc1_ref[...] = (
        jnp.dot(h1, w1_ref[...], preferred_element_type=jnp.float32) * dinv
    )


def _fin_body(degt_ref, h1_ref, aggp_ref, msc1_ref, b1_ref, wout_ref, bout_ref,
              out_ref):
    dinv = _dinv_of(degt_ref)
    agg = aggp_ref[0] + aggp_ref[1] - msc1_ref[...]
    h2 = h1_ref[...] + jnp.maximum(agg * dinv + b1_ref[...], 0.0)
    out_ref[...] = (
        jnp.dot(h2, wout_ref[...], preferred_element_type=jnp.float32)
        + bout_ref[...]
    )


def kernel(x, edge_index, W0, b0, W1, b1, Wout, bout):
    x = x.astype(jnp.float32)
    src3 = edge_index[0].reshape(NW, NCH, K)
    dst3 = edge_index[1].reshape(NW, NCH, K)
    b0r = b0.reshape(1, H)
    b1r = b1.reshape(1, H)
    boutr = bout.reshape(1, 2)

    degp = _deg_kernel_build()(dst3)
    degt = jnp.transpose(degp.reshape(NC, N))

    nblk = N // BM
    msc0 = pl.pallas_call(
        _scale_mm_body,
        grid=(nblk,),
        in_specs=[
            pl.BlockSpec((BM, 2), lambda i: (i, 0)),
            pl.BlockSpec((BM, D), lambda i: (i, 0)),
            pl.BlockSpec((D, H), lambda i: (0, 0)),
        ],
        out_specs=pl.BlockSpec((BM, H), lambda i: (i, 0)),
        out_shape=jax.ShapeDtypeStruct((N, H), jnp.float32),
    )(degt, x, W0)

    agg0 = _agg_kernel_build()(src3, dst3, msc0)

    h1, msc1 = pl.pallas_call(
        _mid_body,
        grid=(nblk,),
        in_specs=[
            pl.BlockSpec((BM, 2), lambda i: (i, 0)),
            pl.BlockSpec((2, BM, H), lambda i: (0, i, 0)),
            pl.BlockSpec((BM, H), lambda i: (i, 0)),
            pl.BlockSpec((1, H), lambda i: (0, 0)),
            pl.BlockSpec((H, H), lambda i: (0, 0)),
        ],
        out_specs=[
            pl.BlockSpec((BM, H), lambda i: (i, 0)),
            pl.BlockSpec((BM, H), lambda i: (i, 0)),
        ],
        out_shape=[
            jax.ShapeDtypeStruct((N, H), jnp.float32),
            jax.ShapeDtypeStruct((N, H), jnp.float32),
        ],
    )(degt, agg0, msc0, b0r, W1)

    agg1 = _agg_kernel_build()(src3, dst3, msc1)

    coords = pl.pallas_call(
        _fin_body,
        grid=(nblk,),
        in_specs=[
            pl.BlockSpec((BM, 2), lambda i: (i, 0)),
            pl.BlockSpec((BM, H), lambda i: (i, 0)),
            pl.BlockSpec((2, BM, H), lambda i: (0, i, 0)),
            pl.BlockSpec((BM, H), lambda i: (i, 0)),
            pl.BlockSpec((1, H), lambda i: (0, 0)),
            pl.BlockSpec((H, 2), lambda i: (0, 0)),
            pl.BlockSpec((1, 2), lambda i: (0, 0)),
        ],
        out_specs=pl.BlockSpec((BM, 2), lambda i: (i, 0)),
        out_shape=jax.ShapeDtypeStruct((N, 2), jnp.float32),
    )(degt, h1, agg1, msc1, b1r, Wout, boutr)

    return coords


# speedup vs baseline: 33.4638x; 1.4281x over previous
import functools

import jax
import jax.numpy as jnp
from jax import lax
from jax.experimental import pallas as pl
from jax.experimental.pallas import tpu as pltpu
from jax.experimental.pallas import tpu_sc as plsc

N = 10000
E = 320000
D = 128
H = 64

NC = 2
NS = 16
NW = NC * NS
EPT = E // NW
K = 80
NCH = EPT // K
RPT = N // NS

BM = 1000


def _sc_mesh():
    return plsc.VectorSubcoreMesh(
        core_axis_name="c", subcore_axis_name="s", num_cores=NC, num_subcores=NS
    )


@functools.cache
def _deg_kernel_build():
    return pl.kernel(
        _deg_body,
        out_type=jax.ShapeDtypeStruct((NC * N,), jnp.float32),
        mesh=_sc_mesh(),
        scratch_types=[
            pltpu.VMEM((NCH, K), jnp.int32),
            pltpu.VMEM((K,), jnp.float32),
            pltpu.VMEM((640,), jnp.float32),
            pltpu.VMEM_SHARED((N,), jnp.float32),
        ],
    )


def _deg_body(dst_hbm, out_hbm, didx, ones, zbuf, acc):
    c = lax.axis_index("c")
    s = lax.axis_index("s")
    wid = c * NS + s
    row0 = s * 624
    for j in range(640 // 16):
        zbuf[pl.ds(j * 16, 16)] = jnp.zeros((16,), jnp.float32)
    for j in range(K // 16):
        ones[pl.ds(j * 16, 16)] = jnp.ones((16,), jnp.float32)
    pltpu.sync_copy(zbuf, acc.at[pl.ds(row0, 640)])
    pltpu.sync_copy(dst_hbm.at[wid], didx)
    plsc.subcore_barrier()

    def body(i, carry):
        pltpu.sync_copy(ones, acc.at[didx.at[i]], add=True)
        return carry

    lax.fori_loop(0, NCH, body, 0)
    plsc.subcore_barrier()
    pltpu.sync_copy(acc.at[pl.ds(row0, 640)], zbuf)
    pltpu.sync_copy(zbuf, out_hbm.at[pl.ds(c * N + row0, 640)])


@functools.cache
def _agg_kernel_build():
    return pl.kernel(
        _agg_body,
        out_type=jax.ShapeDtypeStruct((NC, N, H), jnp.float32),
        mesh=_sc_mesh(),
        scratch_types=[
            pltpu.VMEM((NCH, K), jnp.int32),
            pltpu.VMEM((NCH, K), jnp.int32),
            pltpu.VMEM((2, K, H), jnp.float32),
            pltpu.VMEM_SHARED((N, H), jnp.float32),
            pltpu.SemaphoreType.DMA,
            pltpu.SemaphoreType.DMA,
        ],
        compiler_params=pltpu.CompilerParams(use_tc_tiling_on_sc=False),
    )


def _agg_body(src_hbm, dst_hbm, msc_hbm, out_hbm,
              sidx, didx, rows, acc, sem0, sem1):
    c = lax.axis_index("c")
    s = lax.axis_index("s")
    wid = c * NS + s
    row0 = s * 624

    for j in range(640 // K):
        pltpu.sync_copy(msc_hbm.at[pl.ds(row0 + j * K, K)], rows.at[0])
        pltpu.sync_copy(rows.at[0], acc.at[pl.ds(row0 + j * K, K)])

    pltpu.sync_copy(src_hbm.at[wid], sidx)
    pltpu.sync_copy(dst_hbm.at[wid], didx)
    plsc.subcore_barrier()

    sems = (sem0, sem1)
    pltpu.async_copy(msc_hbm.at[sidx.at[0]], rows.at[0], sem0)

    def _phase(i, b):
        nb = 1 - b

        @pl.when(i + 1 < NCH)
        def _():
            pltpu.async_copy(msc_hbm.at[sidx.at[i + 1]], rows.at[nb], sems[nb])

        pltpu.make_async_copy(msc_hbm.at[sidx.at[i]], rows.at[b],
                              sems[b]).wait()
        pltpu.sync_copy(rows.at[b], acc.at[didx.at[i]], add=True)

    def body(i, carry):
        @pl.when(lax.rem(i, 2) == 0)
        def _():
            _phase(i, 0)

        @pl.when(lax.rem(i, 2) == 1)
        def _():
            _phase(i, 1)

        return carry

    lax.fori_loop(0, NCH, body, 0)
    plsc.subcore_barrier()
    for j in range(640 // K):
        pltpu.sync_copy(acc.at[pl.ds(row0 + j * K, K)], rows.at[0])
        pltpu.sync_copy(rows.at[0], out_hbm.at[c, pl.ds(row0 + j * K, K)])


def _dinv_of(degt_ref):
    d = degt_ref[...]
    return lax.rsqrt(1.0 + d[:, :1] + d[:, 1:2])


def _scale_mm_body(degt_ref, x_ref, w_ref, out_ref):
    dinv = _dinv_of(degt_ref)
    m = jnp.dot(x_ref[...], w_ref[...], preferred_element_type=jnp.float32)
    out_ref[...] = m * dinv


def _mid_body(degt_ref, aggp_ref, msc0_ref, b0_ref, w1_ref, h1_ref, msc1_ref):
    dinv = _dinv_of(degt_ref)
    agg = aggp_ref[0] + aggp_ref[1] - msc0_ref[...]
    h1 = jnp.maximum(agg * dinv + b0_ref[...], 0.0)
    h1_ref[...] = h1
    msc1_ref[...] = (
        jnp.dot(h1, w1_ref[...], preferred_element_type=jnp.float32) * dinv
    )


def _fin_body(degt_ref, h1_ref, aggp_ref, msc1_ref, b1_ref, wout_ref, bout_ref,
              out_ref):
    dinv = _dinv_of(degt_ref)
    agg = aggp_ref[0] + aggp_ref[1] - msc1_ref[...]
    h2 = h1_ref[...] + jnp.maximum(agg * dinv + b1_ref[...], 0.0)
    out_ref[...] = (
        jnp.dot(h2, wout_ref[...], preferred_element_type=jnp.float32)
        + bout_ref[...]
    )


def kernel(x, edge_index, W0, b0, W1, b1, Wout, bout):
    x = x.astype(jnp.float32)
    src3 = edge_index[0].reshape(NW, NCH, K)
    dst3 = edge_index[1].reshape(NW, NCH, K)
    b0r = b0.reshape(1, H)
    b1r = b1.reshape(1, H)
    boutr = bout.reshape(1, 2)

    degp = _deg_kernel_build()(dst3)
    degt = jnp.transpose(degp.reshape(NC, N))

    nblk = N // BM
    msc0 = pl.pallas_call(
        _scale_mm_body,
        grid=(nblk,),
        in_specs=[
            pl.BlockSpec((BM, 2), lambda i: (i, 0)),
            pl.BlockSpec((BM, D), lambda i: (i, 0)),
            pl.BlockSpec((D, H), lambda i: (0, 0)),
        ],
        out_specs=pl.BlockSpec((BM, H), lambda i: (i, 0)),
        out_shape=jax.ShapeDtypeStruct((N, H), jnp.float32),
    )(degt, x, W0)

    agg0 = _agg_kernel_build()(src3, dst3, msc0)

    h1, msc1 = pl.pallas_call(
        _mid_body,
        grid=(nblk,),
        in_specs=[
            pl.BlockSpec((BM, 2), lambda i: (i, 0)),
            pl.BlockSpec((2, BM, H), lambda i: (0, i, 0)),
            pl.BlockSpec((BM, H), lambda i: (i, 0)),
            pl.BlockSpec((1, H), lambda i: (0, 0)),
            pl.BlockSpec((H, H), lambda i: (0, 0)),
        ],
        out_specs=[
            pl.BlockSpec((BM, H), lambda i: (i, 0)),
            pl.BlockSpec((BM, H), lambda i: (i, 0)),
        ],
        out_shape=[
            jax.ShapeDtypeStruct((N, H), jnp.float32),
            jax.ShapeDtypeStruct((N, H), jnp.float32),
        ],
    )(degt, agg0, msc0, b0r, W1)

    agg1 = _agg_kernel_build()(src3, dst3, msc1)

    coords = pl.pallas_call(
        _fin_body,
        grid=(nblk,),
        in_specs=[
            pl.BlockSpec((BM, 2), lambda i: (i, 0)),
            pl.BlockSpec((BM, H), lambda i: (i, 0)),
            pl.BlockSpec((2, BM, H), lambda i: (0, i, 0)),
            pl.BlockSpec((BM, H), lambda i: (i, 0)),
            pl.BlockSpec((1, H), lambda i: (0, 0)),
            pl.BlockSpec((H, 2), lambda i: (0, 0)),
            pl.BlockSpec((1, 2), lambda i: (0, 0)),
        ],
        out_specs=pl.BlockSpec((BM, 2), lambda i: (i, 0)),
        out_shape=jax.ShapeDtypeStruct((N, 2), jnp.float32),
    )(degt, h1, agg1, msc1, b1r, Wout, boutr)

    return coords


# speedup vs baseline: 41.3464x; 1.2356x over previous
import functools

import jax
import jax.numpy as jnp
from jax import lax
from jax.experimental import pallas as pl
from jax.experimental.pallas import tpu as pltpu
from jax.experimental.pallas import tpu_sc as plsc

N = 10000
E = 320000
D = 128
H = 64

NC = 2
NS = 16
NW = NC * NS
EPT = E // NW
K = 80
NCH = EPT // K
RPT = N // NS

BM = 1000


def _sc_mesh():
    return plsc.VectorSubcoreMesh(
        core_axis_name="c", subcore_axis_name="s", num_cores=NC, num_subcores=NS
    )


@functools.cache
def _deg_kernel_build():
    return pl.kernel(
        _deg_body,
        out_type=jax.ShapeDtypeStruct((NC * N,), jnp.float32),
        mesh=_sc_mesh(),
        scratch_types=[
            pltpu.VMEM((NCH, K), jnp.int32),
            pltpu.VMEM((K,), jnp.float32),
            pltpu.VMEM((640,), jnp.float32),
            pltpu.VMEM_SHARED((N,), jnp.float32),
        ],
    )


def _deg_body(dst_hbm, out_hbm, didx, ones, zbuf, acc):
    c = lax.axis_index("c")
    s = lax.axis_index("s")
    wid = c * NS + s
    row0 = s * 624
    for j in range(640 // 16):
        zbuf[pl.ds(j * 16, 16)] = jnp.zeros((16,), jnp.float32)
    for j in range(K // 16):
        ones[pl.ds(j * 16, 16)] = jnp.ones((16,), jnp.float32)
    pltpu.sync_copy(zbuf, acc.at[pl.ds(row0, 640)])
    pltpu.sync_copy(dst_hbm.at[wid], didx)
    plsc.subcore_barrier()

    def body(i, carry):
        pltpu.sync_copy(ones, acc.at[didx.at[i]], add=True)
        return carry

    lax.fori_loop(0, NCH, body, 0)
    plsc.subcore_barrier()
    pltpu.sync_copy(acc.at[pl.ds(row0, 640)], zbuf)
    pltpu.sync_copy(zbuf, out_hbm.at[pl.ds(c * N + row0, 640)])


@functools.cache
def _agg_kernel_build():
    return pl.kernel(
        _agg_body,
        out_type=jax.ShapeDtypeStruct((NC, N, H), jnp.float32),
        mesh=_sc_mesh(),
        scratch_types=[
            pltpu.VMEM((NCH, K), jnp.int32),
            pltpu.VMEM((NCH, K), jnp.int32),
            pltpu.VMEM((4, K, H), jnp.float32),
            pltpu.VMEM_SHARED((N, H), jnp.float32),
            pltpu.SemaphoreType.DMA,
            pltpu.SemaphoreType.DMA,
            pltpu.SemaphoreType.DMA,
            pltpu.SemaphoreType.DMA,
        ],
        compiler_params=pltpu.CompilerParams(use_tc_tiling_on_sc=False),
    )


def _agg_body(src_hbm, dst_hbm, msc_hbm, out_hbm,
              sidx, didx, rows, acc, sem0, sem1, sem2, sem3):
    c = lax.axis_index("c")
    s = lax.axis_index("s")
    wid = c * NS + s
    row0 = s * 624

    for j in range(640 // K):
        pltpu.sync_copy(msc_hbm.at[pl.ds(row0 + j * K, K)], rows.at[0])
        pltpu.sync_copy(rows.at[0], acc.at[pl.ds(row0 + j * K, K)])

    pltpu.sync_copy(src_hbm.at[wid], sidx)
    pltpu.sync_copy(dst_hbm.at[wid], didx)
    plsc.subcore_barrier()

    gsems = (sem0, sem1, sem2, sem3)
    NB = 4
    for b in range(NB - 1):
        pltpu.async_copy(msc_hbm.at[sidx.at[b]], rows.at[b], gsems[b])

    def _phase(i, b):
        fb = (b + NB - 1) % NB

        @pl.when(i + NB - 1 < NCH)
        def _():
            pltpu.async_copy(msc_hbm.at[sidx.at[i + NB - 1]], rows.at[fb],
                             gsems[fb])

        pltpu.make_async_copy(msc_hbm.at[sidx.at[i]], rows.at[b],
                              gsems[b]).wait()
        pltpu.sync_copy(rows.at[b], acc.at[didx.at[i]], add=True)

    def body(i, carry):
        for b in range(NB):
            @pl.when(lax.rem(i, NB) == b)
            def _(b=b):
                _phase(i, b)

        return carry

    lax.fori_loop(0, NCH, body, 0)
    plsc.subcore_barrier()
    for j in range(640 // K):
        pltpu.sync_copy(acc.at[pl.ds(row0 + j * K, K)], rows.at[0])
        pltpu.sync_copy(rows.at[0], out_hbm.at[c, pl.ds(row0 + j * K, K)])


def _dinv_of(degt_ref):
    d = degt_ref[...]
    return lax.rsqrt(1.0 + d[:, :1] + d[:, 1:2])


def _scale_mm_body(degt_ref, x_ref, w_ref, out_ref):
    dinv = _dinv_of(degt_ref)
    m = jnp.dot(x_ref[...], w_ref[...], preferred_element_type=jnp.float32)
    out_ref[...] = m * dinv


def _mid_body(degt_ref, aggp_ref, msc0_ref, b0_ref, w1_ref, h1_ref, msc1_ref):
    dinv = _dinv_of(degt_ref)
    agg = aggp_ref[0] + aggp_ref[1] - msc0_ref[...]
    h1 = jnp.maximum(agg * dinv + b0_ref[...], 0.0)
    h1_ref[...] = h1
    msc1_ref[...] = (
        jnp.dot(h1, w1_ref[...], preferred_element_type=jnp.float32) * dinv
    )


def _fin_body(degt_ref, h1_ref, aggp_ref, msc1_ref, b1_ref, wout_ref, bout_ref,
              out_ref):
    dinv = _dinv_of(degt_ref)
    agg = aggp_ref[0] + aggp_ref[1] - msc1_ref[...]
    h2 = h1_ref[...] + jnp.maximum(agg * dinv + b1_ref[...], 0.0)
    out_ref[...] = (
        jnp.dot(h2, wout_ref[...], preferred_element_type=jnp.float32)
        + bout_ref[...]
    )


def kernel(x, edge_index, W0, b0, W1, b1, Wout, bout):
    x = x.astype(jnp.float32)
    src3 = edge_index[0].reshape(NW, NCH, K)
    dst3 = edge_index[1].reshape(NW, NCH, K)
    b0r = b0.reshape(1, H)
    b1r = b1.reshape(1, H)
    boutr = bout.reshape(1, 2)

    degp = _deg_kernel_build()(dst3)
    degt = jnp.transpose(degp.reshape(NC, N))

    nblk = N // BM
    msc0 = pl.pallas_call(
        _scale_mm_body,
        grid=(nblk,),
        in_specs=[
            pl.BlockSpec((BM, 2), lambda i: (i, 0)),
            pl.BlockSpec((BM, D), lambda i: (i, 0)),
            pl.BlockSpec((D, H), lambda i: (0, 0)),
        ],
        out_specs=pl.BlockSpec((BM, H), lambda i: (i, 0)),
        out_shape=jax.ShapeDtypeStruct((N, H), jnp.float32),
    )(degt, x, W0)

    agg0 = _agg_kernel_build()(src3, dst3, msc0)

    h1, msc1 = pl.pallas_call(
        _mid_body,
        grid=(nblk,),
        in_specs=[
            pl.BlockSpec((BM, 2), lambda i: (i, 0)),
            pl.BlockSpec((2, BM, H), lambda i: (0, i, 0)),
            pl.BlockSpec((BM, H), lambda i: (i, 0)),
            pl.BlockSpec((1, H), lambda i: (0, 0)),
            pl.BlockSpec((H, H), lambda i: (0, 0)),
        ],
        out_specs=[
            pl.BlockSpec((BM, H), lambda i: (i, 0)),
            pl.BlockSpec((BM, H), lambda i: (i, 0)),
        ],
        out_shape=[
            jax.ShapeDtypeStruct((N, H), jnp.float32),
            jax.ShapeDtypeStruct((N, H), jnp.float32),
        ],
    )(degt, agg0, msc0, b0r, W1)

    agg1 = _agg_kernel_build()(src3, dst3, msc1)

    coords = pl.pallas_call(
        _fin_body,
        grid=(nblk,),
        in_specs=[
            pl.BlockSpec((BM, 2), lambda i: (i, 0)),
            pl.BlockSpec((BM, H), lambda i: (i, 0)),
            pl.BlockSpec((2, BM, H), lambda i: (0, i, 0)),
            pl.BlockSpec((BM, H), lambda i: (i, 0)),
            pl.BlockSpec((1, H), lambda i: (0, 0)),
            pl.BlockSpec((H, 2), lambda i: (0, 0)),
            pl.BlockSpec((1, 2), lambda i: (0, 0)),
        ],
        out_specs=pl.BlockSpec((BM, 2), lambda i: (i, 0)),
        out_shape=jax.ShapeDtypeStruct((N, 2), jnp.float32),
    )(degt, h1, agg1, msc1, b1r, Wout, boutr)

    return coords


# speedup vs baseline: 43.7090x; 1.0571x over previous
import functools

import jax
import jax.numpy as jnp
from jax import lax
from jax.experimental import pallas as pl
from jax.experimental.pallas import tpu as pltpu
from jax.experimental.pallas import tpu_sc as plsc

N = 10000
E = 320000
D = 128
H = 64

NC = 2
NS = 16
NW = NC * NS
EPT = E // NW
K = 80
NCH = EPT // K
RPT = N // NS

BM = 1000


def _sc_mesh():
    return plsc.VectorSubcoreMesh(
        core_axis_name="c", subcore_axis_name="s", num_cores=NC, num_subcores=NS
    )


@functools.cache
def _deg_kernel_build():
    return pl.kernel(
        _deg_body,
        out_type=jax.ShapeDtypeStruct((NC * N,), jnp.float32),
        mesh=_sc_mesh(),
        scratch_types=[
            pltpu.VMEM((NCH, K), jnp.int32),
            pltpu.VMEM((K,), jnp.float32),
            pltpu.VMEM((640,), jnp.float32),
            pltpu.VMEM_SHARED((N,), jnp.float32),
        ],
    )


def _deg_body(dst_hbm, out_hbm, didx, ones, zbuf, acc):
    c = lax.axis_index("c")
    s = lax.axis_index("s")
    wid = c * NS + s
    row0 = s * 624
    for j in range(640 // 16):
        zbuf[pl.ds(j * 16, 16)] = jnp.zeros((16,), jnp.float32)
    for j in range(K // 16):
        ones[pl.ds(j * 16, 16)] = jnp.ones((16,), jnp.float32)
    pltpu.sync_copy(zbuf, acc.at[pl.ds(row0, 640)])
    pltpu.sync_copy(dst_hbm.at[wid], didx)
    plsc.subcore_barrier()

    def body(i, carry):
        pltpu.sync_copy(ones, acc.at[didx.at[i]], add=True)
        return carry

    lax.fori_loop(0, NCH, body, 0)
    plsc.subcore_barrier()
    pltpu.sync_copy(acc.at[pl.ds(row0, 640)], zbuf)
    pltpu.sync_copy(zbuf, out_hbm.at[pl.ds(c * N + row0, 640)])


@functools.cache
def _agg_kernel_build():
    return pl.kernel(
        _agg_body,
        out_type=jax.ShapeDtypeStruct((NC, N, H), jnp.float32),
        mesh=_sc_mesh(),
        scratch_types=[
            pltpu.VMEM((NCH, K), jnp.int32),
            pltpu.VMEM((NCH, K), jnp.int32),
            pltpu.VMEM((6, K, H), jnp.float32),
            pltpu.VMEM((320, H), jnp.float32),
            pltpu.VMEM_SHARED((N, H), jnp.float32),
            pltpu.SemaphoreType.DMA,
            pltpu.SemaphoreType.DMA,
            pltpu.SemaphoreType.DMA,
            pltpu.SemaphoreType.DMA,
            pltpu.SemaphoreType.DMA,
            pltpu.SemaphoreType.DMA,
        ],
        compiler_params=pltpu.CompilerParams(use_tc_tiling_on_sc=False),
    )


def _agg_body(src_hbm, dst_hbm, msc_hbm, out_hbm,
              sidx, didx, rows, stage, acc,
              sem0, sem1, sem2, sem3, sem4, sem5):
    c = lax.axis_index("c")
    s = lax.axis_index("s")
    wid = c * NS + s
    row0 = s * 624

    for j in range(2):
        pltpu.sync_copy(msc_hbm.at[pl.ds(row0 + j * 320, 320)], stage)
        pltpu.sync_copy(stage, acc.at[pl.ds(row0 + j * 320, 320)])

    pltpu.sync_copy(src_hbm.at[wid], sidx)
    pltpu.sync_copy(dst_hbm.at[wid], didx)
    plsc.subcore_barrier()

    gsems = (sem0, sem1, sem2, sem3, sem4, sem5)
    NB = 6
    for b in range(NB - 1):
        pltpu.async_copy(msc_hbm.at[sidx.at[b]], rows.at[b], gsems[b])

    def _phase(i, b):
        fb = (b + NB - 1) % NB

        @pl.when(i + NB - 1 < NCH)
        def _():
            pltpu.async_copy(msc_hbm.at[sidx.at[i + NB - 1]], rows.at[fb],
                             gsems[fb])

        pltpu.make_async_copy(msc_hbm.at[sidx.at[i]], rows.at[b],
                              gsems[b]).wait()
        pltpu.sync_copy(rows.at[b], acc.at[didx.at[i]], add=True)

    def body(i, carry):
        for b in range(NB):
            @pl.when(lax.rem(i, NB) == b)
            def _(b=b):
                _phase(i, b)

        return carry

    lax.fori_loop(0, NCH, body, 0)
    plsc.subcore_barrier()
    for j in range(2):
        pltpu.sync_copy(acc.at[pl.ds(row0 + j * 320, 320)], stage)
        pltpu.sync_copy(stage, out_hbm.at[c, pl.ds(row0 + j * 320, 320)])


def _dinv_of(degt_ref):
    d = degt_ref[...]
    return lax.rsqrt(1.0 + d[:, :1] + d[:, 1:2])


def _scale_mm_body(degt_ref, x_ref, w_ref, out_ref):
    dinv = _dinv_of(degt_ref)
    m = jnp.dot(x_ref[...], w_ref[...], preferred_element_type=jnp.float32)
    out_ref[...] = m * dinv


def _mid_body(degt_ref, aggp_ref, msc0_ref, b0_ref, w1_ref, h1_ref, msc1_ref):
    dinv = _dinv_of(degt_ref)
    agg = aggp_ref[0] + aggp_ref[1] - msc0_ref[...]
    h1 = jnp.maximum(agg * dinv + b0_ref[...], 0.0)
    h1_ref[...] = h1
    msc1_ref[...] = (
        jnp.dot(h1, w1_ref[...], preferred_element_type=jnp.float32) * dinv
    )


def _fin_body(degt_ref, h1_ref, aggp_ref, msc1_ref, b1_ref, wout_ref, bout_ref,
              out_ref):
    dinv = _dinv_of(degt_ref)
    agg = aggp_ref[0] + aggp_ref[1] - msc1_ref[...]
    h2 = h1_ref[...] + jnp.maximum(agg * dinv + b1_ref[...], 0.0)
    out_ref[...] = (
        jnp.dot(h2, wout_ref[...], preferred_element_type=jnp.float32)
        + bout_ref[...]
    )


def kernel(x, edge_index, W0, b0, W1, b1, Wout, bout):
    x = x.astype(jnp.float32)
    src3 = edge_index[0].reshape(NW, NCH, K)
    dst3 = edge_index[1].reshape(NW, NCH, K)
    b0r = b0.reshape(1, H)
    b1r = b1.reshape(1, H)
    boutr = bout.reshape(1, 2)

    degp = _deg_kernel_build()(dst3)
    degt = jnp.transpose(degp.reshape(NC, N))

    nblk = N // BM
    msc0 = pl.pallas_call(
        _scale_mm_body,
        grid=(nblk,),
        in_specs=[
            pl.BlockSpec((BM, 2), lambda i: (i, 0)),
            pl.BlockSpec((BM, D), lambda i: (i, 0)),
            pl.BlockSpec((D, H), lambda i: (0, 0)),
        ],
        out_specs=pl.BlockSpec((BM, H), lambda i: (i, 0)),
        out_shape=jax.ShapeDtypeStruct((N, H), jnp.float32),
    )(degt, x, W0)

    agg0 = _agg_kernel_build()(src3, dst3, msc0)

    h1, msc1 = pl.pallas_call(
        _mid_body,
        grid=(nblk,),
        in_specs=[
            pl.BlockSpec((BM, 2), lambda i: (i, 0)),
            pl.BlockSpec((2, BM, H), lambda i: (0, i, 0)),
            pl.BlockSpec((BM, H), lambda i: (i, 0)),
            pl.BlockSpec((1, H), lambda i: (0, 0)),
            pl.BlockSpec((H, H), lambda i: (0, 0)),
        ],
        out_specs=[
            pl.BlockSpec((BM, H), lambda i: (i, 0)),
            pl.BlockSpec((BM, H), lambda i: (i, 0)),
        ],
        out_shape=[
            jax.ShapeDtypeStruct((N, H), jnp.float32),
            jax.ShapeDtypeStruct((N, H), jnp.float32),
        ],
    )(degt, agg0, msc0, b0r, W1)

    agg1 = _agg_kernel_build()(src3, dst3, msc1)

    coords = pl.pallas_call(
        _fin_body,
        grid=(nblk,),
        in_specs=[
            pl.BlockSpec((BM, 2), lambda i: (i, 0)),
            pl.BlockSpec((BM, H), lambda i: (i, 0)),
            pl.BlockSpec((2, BM, H), lambda i: (0, i, 0)),
            pl.BlockSpec((BM, H), lambda i: (i, 0)),
            pl.BlockSpec((1, H), lambda i: (0, 0)),
            pl.BlockSpec((H, 2), lambda i: (0, 0)),
            pl.BlockSpec((1, 2), lambda i: (0, 0)),
        ],
        out_specs=pl.BlockSpec((BM, 2), lambda i: (i, 0)),
        out_shape=jax.ShapeDtypeStruct((N, 2), jnp.float32),
    )(degt, h1, agg1, msc1, b1r, Wout, boutr)

    return coords


# speedup vs baseline: 45.0284x; 1.0302x over previous
import functools

import jax
import jax.numpy as jnp
from jax import lax
from jax.experimental import pallas as pl
from jax.experimental.pallas import tpu as pltpu
from jax.experimental.pallas import tpu_sc as plsc

N = 10000
E = 320000
D = 128
H = 64

NC = 2
NS = 16
NW = NC * NS
EPT = E // NW
K = 80
NCH = EPT // K
RPT = N // NS

BM = 2000


def _sc_mesh():
    return plsc.VectorSubcoreMesh(
        core_axis_name="c", subcore_axis_name="s", num_cores=NC, num_subcores=NS
    )


@functools.cache
def _deg_kernel_build():
    return pl.kernel(
        _deg_body,
        out_type=jax.ShapeDtypeStruct((NC * N,), jnp.float32),
        mesh=_sc_mesh(),
        scratch_types=[
            pltpu.VMEM((NCH, K), jnp.int32),
            pltpu.VMEM((K,), jnp.float32),
            pltpu.VMEM((640,), jnp.float32),
            pltpu.VMEM_SHARED((N,), jnp.float32),
        ],
    )


def _deg_body(dst_hbm, out_hbm, didx, ones, zbuf, acc):
    c = lax.axis_index("c")
    s = lax.axis_index("s")
    wid = c * NS + s
    row0 = s * 624
    for j in range(640 // 16):
        zbuf[pl.ds(j * 16, 16)] = jnp.zeros((16,), jnp.float32)
    for j in range(K // 16):
        ones[pl.ds(j * 16, 16)] = jnp.ones((16,), jnp.float32)
    pltpu.sync_copy(zbuf, acc.at[pl.ds(row0, 640)])
    pltpu.sync_copy(dst_hbm.at[wid], didx)
    plsc.subcore_barrier()

    def body(i, carry):
        pltpu.sync_copy(ones, acc.at[didx.at[i]], add=True)
        return carry

    lax.fori_loop(0, NCH, body, 0)
    plsc.subcore_barrier()
    pltpu.sync_copy(acc.at[pl.ds(row0, 640)], zbuf)
    pltpu.sync_copy(zbuf, out_hbm.at[pl.ds(c * N + row0, 640)])


@functools.cache
def _agg_kernel_build():
    return pl.kernel(
        _agg_body,
        out_type=jax.ShapeDtypeStruct((NC, N, H), jnp.float32),
        mesh=_sc_mesh(),
        scratch_types=[
            pltpu.VMEM((NCH, K), jnp.int32),
            pltpu.VMEM((NCH, K), jnp.int32),
            pltpu.VMEM((6, K, H), jnp.float32),
            pltpu.VMEM((320, H), jnp.float32),
            pltpu.VMEM_SHARED((N, H), jnp.float32),
            pltpu.SemaphoreType.DMA,
            pltpu.SemaphoreType.DMA,
            pltpu.SemaphoreType.DMA,
            pltpu.SemaphoreType.DMA,
            pltpu.SemaphoreType.DMA,
            pltpu.SemaphoreType.DMA,
        ],
        compiler_params=pltpu.CompilerParams(use_tc_tiling_on_sc=False),
    )


def _agg_body(src_hbm, dst_hbm, msc_hbm, out_hbm,
              sidx, didx, rows, stage, acc,
              sem0, sem1, sem2, sem3, sem4, sem5):
    c = lax.axis_index("c")
    s = lax.axis_index("s")
    wid = c * NS + s
    row0 = s * 624

    for j in range(2):
        pltpu.sync_copy(msc_hbm.at[pl.ds(row0 + j * 320, 320)], stage)
        pltpu.sync_copy(stage, acc.at[pl.ds(row0 + j * 320, 320)])

    pltpu.sync_copy(src_hbm.at[wid], sidx)
    pltpu.sync_copy(dst_hbm.at[wid], didx)
    plsc.subcore_barrier()

    gsems = (sem0, sem1, sem2, sem3, sem4, sem5)
    NB = 6
    for b in range(NB - 1):
        pltpu.async_copy(msc_hbm.at[sidx.at[b]], rows.at[b], gsems[b])

    def _phase(i, b):
        fb = (b + NB - 1) % NB

        @pl.when(i + NB - 1 < NCH)
        def _():
            pltpu.async_copy(msc_hbm.at[sidx.at[i + NB - 1]], rows.at[fb],
                             gsems[fb])

        pltpu.make_async_copy(msc_hbm.at[sidx.at[i]], rows.at[b],
                              gsems[b]).wait()
        pltpu.sync_copy(rows.at[b], acc.at[didx.at[i]], add=True)

    def body(i, carry):
        for b in range(NB):
            @pl.when(lax.rem(i, NB) == b)
            def _(b=b):
                _phase(i, b)

        return carry

    lax.fori_loop(0, NCH, body, 0)
    plsc.subcore_barrier()
    for j in range(2):
        pltpu.sync_copy(acc.at[pl.ds(row0 + j * 320, 320)], stage)
        pltpu.sync_copy(stage, out_hbm.at[c, pl.ds(row0 + j * 320, 320)])


def _dinv_of(degt_ref):
    d = degt_ref[...]
    return lax.rsqrt(1.0 + d[:, :1] + d[:, 1:2])


def _scale_mm_body(degt_ref, x_ref, w_ref, out_ref):
    dinv = _dinv_of(degt_ref)
    m = jnp.dot(x_ref[...], w_ref[...], preferred_element_type=jnp.float32)
    out_ref[...] = m * dinv


def _mid_body(degt_ref, aggp_ref, msc0_ref, b0_ref, w1_ref, h1_ref, msc1_ref):
    dinv = _dinv_of(degt_ref)
    agg = aggp_ref[0] + aggp_ref[1] - msc0_ref[...]
    h1 = jnp.maximum(agg * dinv + b0_ref[...], 0.0)
    h1_ref[...] = h1
    msc1_ref[...] = (
        jnp.dot(h1, w1_ref[...], preferred_element_type=jnp.float32) * dinv
    )


def _fin_body(degt_ref, h1_ref, aggp_ref, msc1_ref, b1_ref, wout_ref, bout_ref,
              out_ref):
    dinv = _dinv_of(degt_ref)
    agg = aggp_ref[0] + aggp_ref[1] - msc1_ref[...]
    h2 = h1_ref[...] + jnp.maximum(agg * dinv + b1_ref[...], 0.0)
    out_ref[...] = (
        jnp.dot(h2, wout_ref[...], preferred_element_type=jnp.float32)
        + bout_ref[...]
    )


def kernel(x, edge_index, W0, b0, W1, b1, Wout, bout):
    x = x.astype(jnp.float32)
    src3 = edge_index[0].reshape(NW, NCH, K)
    dst3 = edge_index[1].reshape(NW, NCH, K)
    b0r = b0.reshape(1, H)
    b1r = b1.reshape(1, H)
    boutr = bout.reshape(1, 2)

    degp = _deg_kernel_build()(dst3)
    degt = jnp.transpose(degp.reshape(NC, N))

    nblk = N // BM
    msc0 = pl.pallas_call(
        _scale_mm_body,
        grid=(nblk,),
        in_specs=[
            pl.BlockSpec((BM, 2), lambda i: (i, 0)),
            pl.BlockSpec((BM, D), lambda i: (i, 0)),
            pl.BlockSpec((D, H), lambda i: (0, 0)),
        ],
        out_specs=pl.BlockSpec((BM, H), lambda i: (i, 0)),
        out_shape=jax.ShapeDtypeStruct((N, H), jnp.float32),
    )(degt, x, W0)

    agg0 = _agg_kernel_build()(src3, dst3, msc0)

    h1, msc1 = pl.pallas_call(
        _mid_body,
        grid=(nblk,),
        in_specs=[
            pl.BlockSpec((BM, 2), lambda i: (i, 0)),
            pl.BlockSpec((2, BM, H), lambda i: (0, i, 0)),
            pl.BlockSpec((BM, H), lambda i: (i, 0)),
            pl.BlockSpec((1, H), lambda i: (0, 0)),
            pl.BlockSpec((H, H), lambda i: (0, 0)),
        ],
        out_specs=[
            pl.BlockSpec((BM, H), lambda i: (i, 0)),
            pl.BlockSpec((BM, H), lambda i: (i, 0)),
        ],
        out_shape=[
            jax.ShapeDtypeStruct((N, H), jnp.float32),
            jax.ShapeDtypeStruct((N, H), jnp.float32),
        ],
    )(degt, agg0, msc0, b0r, W1)

    agg1 = _agg_kernel_build()(src3, dst3, msc1)

    coords = pl.pallas_call(
        _fin_body,
        grid=(nblk,),
        in_specs=[
            pl.BlockSpec((BM, 2), lambda i: (i, 0)),
            pl.BlockSpec((BM, H), lambda i: (i, 0)),
            pl.BlockSpec((2, BM, H), lambda i: (0, i, 0)),
            pl.BlockSpec((BM, H), lambda i: (i, 0)),
            pl.BlockSpec((1, H), lambda i: (0, 0)),
            pl.BlockSpec((H, 2), lambda i: (0, 0)),
            pl.BlockSpec((1, 2), lambda i: (0, 0)),
        ],
        out_specs=pl.BlockSpec((BM, 2), lambda i: (i, 0)),
        out_shape=jax.ShapeDtypeStruct((N, 2), jnp.float32),
    )(degt, h1, agg1, msc1, b1r, Wout, boutr)

    return coords


# speedup vs baseline: 45.1780x; 1.0033x over previous
import functools

import jax
import jax.numpy as jnp
from jax import lax
from jax.experimental import pallas as pl
from jax.experimental.pallas import tpu as pltpu
from jax.experimental.pallas import tpu_sc as plsc

N = 10000
E = 320000
D = 128
H = 64

NC = 2
NS = 16
NW = NC * NS
EPT = E // NW
K = 80
NCH = EPT // K
RPT = N // NS

BM = 2000


def _sc_mesh():
    return plsc.VectorSubcoreMesh(
        core_axis_name="c", subcore_axis_name="s", num_cores=NC, num_subcores=NS
    )


@functools.cache
def _deg_kernel_build():
    return pl.kernel(
        _deg_body,
        out_type=jax.ShapeDtypeStruct((NC * N,), jnp.float32),
        mesh=_sc_mesh(),
        scratch_types=[
            pltpu.VMEM((NCH, K), jnp.int32),
            pltpu.VMEM((K,), jnp.float32),
            pltpu.VMEM((640,), jnp.float32),
            pltpu.VMEM_SHARED((N,), jnp.float32),
        ],
    )


def _deg_body(dst_hbm, out_hbm, didx, ones, zbuf, acc):
    c = lax.axis_index("c")
    s = lax.axis_index("s")
    wid = c * NS + s
    row0 = s * 624
    for j in range(640 // 16):
        zbuf[pl.ds(j * 16, 16)] = jnp.zeros((16,), jnp.float32)
    for j in range(K // 16):
        ones[pl.ds(j * 16, 16)] = jnp.ones((16,), jnp.float32)
    pltpu.sync_copy(zbuf, acc.at[pl.ds(row0, 640)])
    pltpu.sync_copy(dst_hbm.at[wid], didx)
    plsc.subcore_barrier()

    def body(i, carry):
        pltpu.sync_copy(ones, acc.at[didx.at[i]], add=True)
        return carry

    lax.fori_loop(0, NCH, body, 0)
    plsc.subcore_barrier()
    pltpu.sync_copy(acc.at[pl.ds(row0, 640)], zbuf)
    pltpu.sync_copy(zbuf, out_hbm.at[pl.ds(c * N + row0, 640)])


@functools.cache
def _agg_kernel_build():
    return pl.kernel(
        _agg_body,
        out_type=jax.ShapeDtypeStruct((NC, N, H), jnp.float32),
        mesh=_sc_mesh(),
        scratch_types=[
            pltpu.VMEM((NCH, K), jnp.int32),
            pltpu.VMEM((NCH, K), jnp.int32),
            pltpu.VMEM((6, K, H), jnp.float32),
            pltpu.VMEM((320, H), jnp.float32),
            pltpu.VMEM_SHARED((N, H), jnp.float32),
            pltpu.SemaphoreType.DMA,
            pltpu.SemaphoreType.DMA,
            pltpu.SemaphoreType.DMA,
            pltpu.SemaphoreType.DMA,
            pltpu.SemaphoreType.DMA,
            pltpu.SemaphoreType.DMA,
        ],
        compiler_params=pltpu.CompilerParams(use_tc_tiling_on_sc=False),
    )


def _agg_body(src_hbm, dst_hbm, msc_hbm, out_hbm,
              sidx, didx, rows, stage, acc,
              sem0, sem1, sem2, sem3, sem4, sem5):
    c = lax.axis_index("c")
    s = lax.axis_index("s")
    wid = c * NS + s
    row0 = s * 624

    for j in range(2):
        pltpu.sync_copy(msc_hbm.at[pl.ds(row0 + j * 320, 320)], stage)
        pltpu.sync_copy(stage, acc.at[pl.ds(row0 + j * 320, 320)])

    pltpu.sync_copy(src_hbm.at[wid], sidx)
    pltpu.sync_copy(dst_hbm.at[wid], didx)
    plsc.subcore_barrier()

    gsems = (sem0, sem1, sem2, sem3, sem4, sem5)
    NB = 6
    for b in range(NB - 1):
        pltpu.async_copy(msc_hbm.at[sidx.at[b]], rows.at[b], gsems[b])

    def _phase(i, b):
        fb = (b + NB - 1) % NB

        @pl.when(i + NB - 1 < NCH)
        def _():
            pltpu.async_copy(msc_hbm.at[sidx.at[i + NB - 1]], rows.at[fb],
                             gsems[fb])

        pltpu.make_async_copy(msc_hbm.at[sidx.at[i]], rows.at[b],
                              gsems[b]).wait()
        pltpu.sync_copy(rows.at[b], acc.at[didx.at[i]], add=True)

    def body(i, carry):
        for b in range(NB):
            @pl.when(lax.rem(i, NB) == b)
            def _(b=b):
                _phase(i, b)

        return carry

    lax.fori_loop(0, NCH, body, 0)
    plsc.subcore_barrier()
    for j in range(2):
        pltpu.sync_copy(acc.at[pl.ds(row0 + j * 320, 320)], stage)
        pltpu.sync_copy(stage, out_hbm.at[c, pl.ds(row0 + j * 320, 320)])


def _dinv_of(degt_ref):
    d = degt_ref[...]
    return lax.rsqrt(1.0 + d[:, :1] + d[:, 1:2])


def _mm_body(x_ref, w_ref, out_ref):
    out_ref[...] = jnp.dot(x_ref[...], w_ref[...],
                           preferred_element_type=jnp.float32)


def _scale_body(degt_ref, m_ref, out_ref):
    out_ref[...] = m_ref[...] * _dinv_of(degt_ref)


def _mid_body(degt_ref, aggp_ref, msc0_ref, b0_ref, w1_ref, h1_ref, msc1_ref):
    dinv = _dinv_of(degt_ref)
    agg = aggp_ref[0] + aggp_ref[1] - msc0_ref[...]
    h1 = jnp.maximum(agg * dinv + b0_ref[...], 0.0)
    h1_ref[...] = h1
    msc1_ref[...] = (
        jnp.dot(h1, w1_ref[...], preferred_element_type=jnp.float32) * dinv
    )


def _fin_body(degt_ref, h1_ref, aggp_ref, msc1_ref, b1_ref, wout_ref, bout_ref,
              out_ref):
    dinv = _dinv_of(degt_ref)
    agg = aggp_ref[0] + aggp_ref[1] - msc1_ref[...]
    h2 = h1_ref[...] + jnp.maximum(agg * dinv + b1_ref[...], 0.0)
    out_ref[...] = (
        jnp.dot(h2, wout_ref[...], preferred_element_type=jnp.float32)
        + bout_ref[...]
    )


def kernel(x, edge_index, W0, b0, W1, b1, Wout, bout):
    x = x.astype(jnp.float32)
    src3 = edge_index[0].reshape(NW, NCH, K)
    dst3 = edge_index[1].reshape(NW, NCH, K)
    b0r = b0.reshape(1, H)
    b1r = b1.reshape(1, H)
    boutr = bout.reshape(1, 2)

    degp = _deg_kernel_build()(dst3)
    degt = jnp.transpose(degp.reshape(NC, N))

    nblk = N // BM
    m0 = pl.pallas_call(
        _mm_body,
        grid=(nblk,),
        in_specs=[
            pl.BlockSpec((BM, D), lambda i: (i, 0)),
            pl.BlockSpec((D, H), lambda i: (0, 0)),
        ],
        out_specs=pl.BlockSpec((BM, H), lambda i: (i, 0)),
        out_shape=jax.ShapeDtypeStruct((N, H), jnp.float32),
    )(x, W0)

    msc0 = pl.pallas_call(
        _scale_body,
        grid=(nblk,),
        in_specs=[
            pl.BlockSpec((BM, 2), lambda i: (i, 0)),
            pl.BlockSpec((BM, H), lambda i: (i, 0)),
        ],
        out_specs=pl.BlockSpec((BM, H), lambda i: (i, 0)),
        out_shape=jax.ShapeDtypeStruct((N, H), jnp.float32),
    )(degt, m0)

    agg0 = _agg_kernel_build()(src3, dst3, msc0)

    h1, msc1 = pl.pallas_call(
        _mid_body,
        grid=(nblk,),
        in_specs=[
            pl.BlockSpec((BM, 2), lambda i: (i, 0)),
            pl.BlockSpec((2, BM, H), lambda i: (0, i, 0)),
            pl.BlockSpec((BM, H), lambda i: (i, 0)),
            pl.BlockSpec((1, H), lambda i: (0, 0)),
            pl.BlockSpec((H, H), lambda i: (0, 0)),
        ],
        out_specs=[
            pl.BlockSpec((BM, H), lambda i: (i, 0)),
            pl.BlockSpec((BM, H), lambda i: (i, 0)),
        ],
        out_shape=[
            jax.ShapeDtypeStruct((N, H), jnp.float32),
            jax.ShapeDtypeStruct((N, H), jnp.float32),
        ],
    )(degt, agg0, msc0, b0r, W1)

    agg1 = _agg_kernel_build()(src3, dst3, msc1)

    coords = pl.pallas_call(
        _fin_body,
        grid=(nblk,),
        in_specs=[
            pl.BlockSpec((BM, 2), lambda i: (i, 0)),
            pl.BlockSpec((BM, H), lambda i: (i, 0)),
            pl.BlockSpec((2, BM, H), lambda i: (0, i, 0)),
            pl.BlockSpec((BM, H), lambda i: (i, 0)),
            pl.BlockSpec((1, H), lambda i: (0, 0)),
            pl.BlockSpec((H, 2), lambda i: (0, 0)),
            pl.BlockSpec((1, 2), lambda i: (0, 0)),
        ],
        out_specs=pl.BlockSpec((BM, 2), lambda i: (i, 0)),
        out_shape=jax.ShapeDtypeStruct((N, 2), jnp.float32),
    )(degt, h1, agg1, msc1, b1r, Wout, boutr)

    return coords


# speedup vs baseline: 49.0922x; 1.0866x over previous
import functools

import jax
import jax.numpy as jnp
from jax import lax
from jax.experimental import pallas as pl
from jax.experimental.pallas import tpu as pltpu
from jax.experimental.pallas import tpu_sc as plsc

N = 10000
E = 320000
D = 128
H = 64

NC = 2
NS = 16
NW = NC * NS
EPT = E // NW
K = 80
NCH = EPT // K
RPT = N // NS

BM = 2000


def _sc_mesh():
    return plsc.VectorSubcoreMesh(
        core_axis_name="c", subcore_axis_name="s", num_cores=NC, num_subcores=NS
    )


@functools.cache
def _deg_kernel_build():
    return pl.kernel(
        _deg_body,
        out_type=jax.ShapeDtypeStruct((NC * N,), jnp.float32),
        mesh=_sc_mesh(),
        scratch_types=[
            pltpu.VMEM((NCH, K), jnp.int32),
            pltpu.VMEM((K,), jnp.float32),
            pltpu.VMEM((640,), jnp.float32),
            pltpu.VMEM_SHARED((N,), jnp.float32),
        ],
    )


def _deg_body(dst_hbm, out_hbm, didx, ones, zbuf, acc):
    c = lax.axis_index("c")
    s = lax.axis_index("s")
    wid = c * NS + s
    row0 = s * 624
    for j in range(640 // 16):
        zbuf[pl.ds(j * 16, 16)] = jnp.zeros((16,), jnp.float32)
    for j in range(K // 16):
        ones[pl.ds(j * 16, 16)] = jnp.ones((16,), jnp.float32)
    pltpu.sync_copy(zbuf, acc.at[pl.ds(row0, 640)])
    pltpu.sync_copy(dst_hbm.at[wid], didx)
    plsc.subcore_barrier()

    def body(i, carry):
        pltpu.sync_copy(ones, acc.at[didx.at[i]], add=True)
        return carry

    lax.fori_loop(0, NCH, body, 0)
    plsc.subcore_barrier()
    pltpu.sync_copy(acc.at[pl.ds(row0, 640)], zbuf)
    pltpu.sync_copy(zbuf, out_hbm.at[pl.ds(c * N + row0, 640)])


@functools.cache
def _agg_kernel_build():
    return pl.kernel(
        _agg_body,
        out_type=jax.ShapeDtypeStruct((NC, N, 128), jnp.float32),
        mesh=_sc_mesh(),
        scratch_types=[
            pltpu.VMEM((NCH, K), jnp.int32),
            pltpu.VMEM((NCH, K), jnp.int32),
            pltpu.VMEM((6, K, H), jnp.float32),
            pltpu.VMEM((320, H), jnp.float32),
            pltpu.VMEM_SHARED((N, H), jnp.float32),
            pltpu.SemaphoreType.DMA,
            pltpu.SemaphoreType.DMA,
            pltpu.SemaphoreType.DMA,
            pltpu.SemaphoreType.DMA,
            pltpu.SemaphoreType.DMA,
            pltpu.SemaphoreType.DMA,
        ],
        compiler_params=pltpu.CompilerParams(use_tc_tiling_on_sc=False),
    )


def _agg_body(src_hbm, dst_hbm, msc_hbm, out_hbm,
              sidx, didx, rows, stage, acc,
              sem0, sem1, sem2, sem3, sem4, sem5):
    c = lax.axis_index("c")
    s = lax.axis_index("s")
    wid = c * NS + s
    row0 = s * 624

    for j in range(2):
        pltpu.sync_copy(msc_hbm.at[pl.ds(row0 + j * 320, 320)], stage)
        pltpu.sync_copy(stage, acc.at[pl.ds(row0 + j * 320, 320)])

    pltpu.sync_copy(src_hbm.at[wid], sidx)
    pltpu.sync_copy(dst_hbm.at[wid], didx)
    plsc.subcore_barrier()

    gsems = (sem0, sem1, sem2, sem3, sem4, sem5)
    NB = 6
    for b in range(NB - 1):
        pltpu.async_copy(msc_hbm.at[sidx.at[b]], rows.at[b], gsems[b])

    def _phase(i, b):
        fb = (b + NB - 1) % NB

        @pl.when(i + NB - 1 < NCH)
        def _():
            pltpu.async_copy(msc_hbm.at[sidx.at[i + NB - 1]], rows.at[fb],
                             gsems[fb])

        pltpu.make_async_copy(msc_hbm.at[sidx.at[i]], rows.at[b],
                              gsems[b]).wait()
        pltpu.sync_copy(rows.at[b], acc.at[didx.at[i]], add=True)

    def body(i, carry):
        for b in range(NB):
            @pl.when(lax.rem(i, NB) == b)
            def _(b=b):
                _phase(i, b)

        return carry

    lax.fori_loop(0, NCH, body, 0)
    plsc.subcore_barrier()
    for j in range(2):
        pltpu.sync_copy(acc.at[pl.ds(row0 + j * 320, 320)], stage)
        pltpu.sync_copy(stage,
                        out_hbm.at[c, pl.ds(row0 + j * 320, 320), pl.ds(0, H)])


def _dinv_of(degt_ref):
    d = degt_ref[...]
    return lax.rsqrt(1.0 + d[:, :1] + d[:, 1:2])


def _mm_body(x_ref, w_ref, out_ref):
    out_ref[...] = jnp.dot(x_ref[...], w_ref[...],
                           preferred_element_type=jnp.float32)


def _scale_body(degt_ref, m_ref, out_ref):
    out_ref[...] = m_ref[...] * _dinv_of(degt_ref)


def _mid_body(degt_ref, aggp_ref, msc0_ref, b0_ref, w1_ref, h1_ref, msc1_ref):
    dinv = _dinv_of(degt_ref)
    agg = aggp_ref[0, :, :H] + aggp_ref[1, :, :H] - msc0_ref[...]
    h1 = jnp.maximum(agg * dinv + b0_ref[...], 0.0)
    h1_ref[...] = h1
    msc1_ref[...] = (
        jnp.dot(h1, w1_ref[...], preferred_element_type=jnp.float32) * dinv
    )


def _fin_body(degt_ref, h1_ref, aggp_ref, msc1_ref, b1_ref, wout_ref, bout_ref,
              out_ref):
    dinv = _dinv_of(degt_ref)
    agg = aggp_ref[0, :, :H] + aggp_ref[1, :, :H] - msc1_ref[...]
    h2 = h1_ref[...] + jnp.maximum(agg * dinv + b1_ref[...], 0.0)
    out_ref[...] = (
        jnp.dot(h2, wout_ref[...], preferred_element_type=jnp.float32)
        + bout_ref[...]
    )


def kernel(x, edge_index, W0, b0, W1, b1, Wout, bout):
    x = x.astype(jnp.float32)
    src3 = edge_index[0].reshape(NW, NCH, K)
    dst3 = edge_index[1].reshape(NW, NCH, K)
    b0r = b0.reshape(1, H)
    b1r = b1.reshape(1, H)
    boutr = bout.reshape(1, 2)

    degp = _deg_kernel_build()(dst3)
    degt = jnp.transpose(degp.reshape(NC, N))

    nblk = N // BM
    m0 = pl.pallas_call(
        _mm_body,
        grid=(nblk,),
        in_specs=[
            pl.BlockSpec((BM, D), lambda i: (i, 0)),
            pl.BlockSpec((D, H), lambda i: (0, 0)),
        ],
        out_specs=pl.BlockSpec((BM, H), lambda i: (i, 0)),
        out_shape=jax.ShapeDtypeStruct((N, H), jnp.float32),
    )(x, W0)

    msc0 = pl.pallas_call(
        _scale_body,
        grid=(nblk,),
        in_specs=[
            pl.BlockSpec((BM, 2), lambda i: (i, 0)),
            pl.BlockSpec((BM, H), lambda i: (i, 0)),
        ],
        out_specs=pl.BlockSpec((BM, H), lambda i: (i, 0)),
        out_shape=jax.ShapeDtypeStruct((N, H), jnp.float32),
    )(degt, m0)

    agg0 = _agg_kernel_build()(src3, dst3, msc0)

    h1, msc1 = pl.pallas_call(
        _mid_body,
        grid=(nblk,),
        in_specs=[
            pl.BlockSpec((BM, 2), lambda i: (i, 0)),
            pl.BlockSpec((2, BM, 128), lambda i: (0, i, 0)),
            pl.BlockSpec((BM, H), lambda i: (i, 0)),
            pl.BlockSpec((1, H), lambda i: (0, 0)),
            pl.BlockSpec((H, H), lambda i: (0, 0)),
        ],
        out_specs=[
            pl.BlockSpec((BM, H), lambda i: (i, 0)),
            pl.BlockSpec((BM, H), lambda i: (i, 0)),
        ],
        out_shape=[
            jax.ShapeDtypeStruct((N, H), jnp.float32),
            jax.ShapeDtypeStruct((N, H), jnp.float32),
        ],
    )(degt, agg0, msc0, b0r, W1)

    agg1 = _agg_kernel_build()(src3, dst3, msc1)

    coords = pl.pallas_call(
        _fin_body,
        grid=(nblk,),
        in_specs=[
            pl.BlockSpec((BM, 2), lambda i: (i, 0)),
            pl.BlockSpec((BM, H), lambda i: (i, 0)),
            pl.BlockSpec((2, BM, 128), lambda i: (0, i, 0)),
            pl.BlockSpec((BM, H), lambda i: (i, 0)),
            pl.BlockSpec((1, H), lambda i: (0, 0)),
            pl.BlockSpec((H, 2), lambda i: (0, 0)),
            pl.BlockSpec((1, 2), lambda i: (0, 0)),
        ],
        out_specs=pl.BlockSpec((BM, 2), lambda i: (i, 0)),
        out_shape=jax.ShapeDtypeStruct((N, 2), jnp.float32),
    )(degt, h1, agg1, msc1, b1r, Wout, boutr)

    return coords


# speedup vs baseline: 49.2235x; 1.0027x over previous
import functools

import jax
import jax.numpy as jnp
from jax import lax
from jax.experimental import pallas as pl
from jax.experimental.pallas import tpu as pltpu
from jax.experimental.pallas import tpu_sc as plsc

N = 10000
E = 320000
D = 128
H = 64

NC = 2
NS = 16
NW = NC * NS
EPT = E // NW
K = 80
NCH = EPT // K
RPT = N // NS

BM = 2000


def _sc_mesh():
    return plsc.VectorSubcoreMesh(
        core_axis_name="c", subcore_axis_name="s", num_cores=NC, num_subcores=NS
    )


@functools.cache
def _deg_kernel_build():
    return pl.kernel(
        _deg_body,
        out_type=jax.ShapeDtypeStruct((NC * N,), jnp.float32),
        mesh=_sc_mesh(),
        scratch_types=[
            pltpu.VMEM((NCH, K), jnp.int32),
            pltpu.VMEM((K,), jnp.float32),
            pltpu.VMEM((640,), jnp.float32),
            pltpu.VMEM_SHARED((N,), jnp.float32),
        ],
    )


def _deg_body(dst_hbm, out_hbm, didx, ones, zbuf, acc):
    c = lax.axis_index("c")
    s = lax.axis_index("s")
    wid = c * NS + s
    row0 = s * 624
    for j in range(640 // 16):
        zbuf[pl.ds(j * 16, 16)] = jnp.zeros((16,), jnp.float32)
    for j in range(K // 16):
        ones[pl.ds(j * 16, 16)] = jnp.ones((16,), jnp.float32)
    pltpu.sync_copy(zbuf, acc.at[pl.ds(row0, 640)])
    pltpu.sync_copy(dst_hbm.at[wid], didx)
    plsc.subcore_barrier()

    def body(i, carry):
        pltpu.sync_copy(ones, acc.at[didx.at[i]], add=True)
        return carry

    lax.fori_loop(0, NCH, body, 0)
    plsc.subcore_barrier()
    pltpu.sync_copy(acc.at[pl.ds(row0, 640)], zbuf)
    pltpu.sync_copy(zbuf, out_hbm.at[pl.ds(c * N + row0, 640)])


@functools.cache
def _agg_kernel_build():
    return pl.kernel(
        _agg_body,
        out_type=jax.ShapeDtypeStruct((NC, N, 128), jnp.float32),
        mesh=_sc_mesh(),
        scratch_types=[
            pltpu.VMEM((NCH, K), jnp.int32),
            pltpu.VMEM((NCH, K), jnp.int32),
            pltpu.VMEM((6, K, H), jnp.float32),
            pltpu.VMEM((320, H), jnp.float32),
            pltpu.VMEM_SHARED((N, H), jnp.float32),
            pltpu.SemaphoreType.DMA,
            pltpu.SemaphoreType.DMA,
            pltpu.SemaphoreType.DMA,
            pltpu.SemaphoreType.DMA,
            pltpu.SemaphoreType.DMA,
            pltpu.SemaphoreType.DMA,
        ],
        compiler_params=pltpu.CompilerParams(use_tc_tiling_on_sc=False),
    )


def _agg_body(src_hbm, dst_hbm, msc_hbm, zrows_hbm, out_hbm,
              sidx, didx, rows, stage, acc,
              sem0, sem1, sem2, sem3, sem4, sem5):
    c = lax.axis_index("c")
    s = lax.axis_index("s")
    wid = c * NS + s
    row0 = s * 624

    @pl.when(c == 0)
    def _():
        for j in range(2):
            pltpu.sync_copy(msc_hbm.at[pl.ds(row0 + j * 320, 320)], stage)
            pltpu.sync_copy(stage, acc.at[pl.ds(row0 + j * 320, 320)])

    @pl.when(c != 0)
    def _():
        pltpu.sync_copy(zrows_hbm, stage)
        for j in range(2):
            pltpu.sync_copy(stage, acc.at[pl.ds(row0 + j * 320, 320)])

    pltpu.sync_copy(src_hbm.at[wid], sidx)
    pltpu.sync_copy(dst_hbm.at[wid], didx)
    plsc.subcore_barrier()

    gsems = (sem0, sem1, sem2, sem3, sem4, sem5)
    NB = 6
    for b in range(NB - 1):
        pltpu.async_copy(msc_hbm.at[sidx.at[b]], rows.at[b], gsems[b])

    def _phase(i, b):
        fb = (b + NB - 1) % NB

        @pl.when(i + NB - 1 < NCH)
        def _():
            pltpu.async_copy(msc_hbm.at[sidx.at[i + NB - 1]], rows.at[fb],
                             gsems[fb])

        pltpu.make_async_copy(msc_hbm.at[sidx.at[i]], rows.at[b],
                              gsems[b]).wait()
        pltpu.sync_copy(rows.at[b], acc.at[didx.at[i]], add=True)

    def body(i, carry):
        for b in range(NB):
            @pl.when(lax.rem(i, NB) == b)
            def _(b=b):
                _phase(i, b)

        return carry

    lax.fori_loop(0, NCH, body, 0)
    plsc.subcore_barrier()
    for j in range(2):
        pltpu.sync_copy(acc.at[pl.ds(row0 + j * 320, 320)], stage)
        pltpu.sync_copy(stage,
                        out_hbm.at[c, pl.ds(row0 + j * 320, 320), pl.ds(0, H)])


def _dinv_of(degt_ref):
    d = degt_ref[...]
    return lax.rsqrt(1.0 + d[:, :1] + d[:, 1:2])


def _mm_body(x_ref, w_ref, out_ref):
    out_ref[...] = jnp.dot(x_ref[...], w_ref[...],
                           preferred_element_type=jnp.float32)


def _scale_body(degt_ref, m_ref, out_ref):
    out_ref[...] = m_ref[...] * _dinv_of(degt_ref)


def _mid_body(degt_ref, aggp_ref, b0_ref, w1_ref, h1_ref, msc1_ref):
    dinv = _dinv_of(degt_ref)
    agg = aggp_ref[0, :, :H] + aggp_ref[1, :, :H]
    h1 = jnp.maximum(agg * dinv + b0_ref[...], 0.0)
    h1_ref[...] = h1
    msc1_ref[...] = (
        jnp.dot(h1, w1_ref[...], preferred_element_type=jnp.float32) * dinv
    )


def _fin_body(degt_ref, h1_ref, aggp_ref, b1_ref, wout_ref, bout_ref,
              out_ref):
    dinv = _dinv_of(degt_ref)
    agg = aggp_ref[0, :, :H] + aggp_ref[1, :, :H]
    h2 = h1_ref[...] + jnp.maximum(agg * dinv + b1_ref[...], 0.0)
    out_ref[...] = (
        jnp.dot(h2, wout_ref[...], preferred_element_type=jnp.float32)
        + bout_ref[...]
    )


def kernel(x, edge_index, W0, b0, W1, b1, Wout, bout):
    x = x.astype(jnp.float32)
    src3 = edge_index[0].reshape(NW, NCH, K)
    dst3 = edge_index[1].reshape(NW, NCH, K)
    b0r = b0.reshape(1, H)
    b1r = b1.reshape(1, H)
    boutr = bout.reshape(1, 2)

    degp = _deg_kernel_build()(dst3)
    degt = jnp.transpose(degp.reshape(NC, N))

    nblk = N // BM
    m0 = pl.pallas_call(
        _mm_body,
        grid=(nblk,),
        in_specs=[
            pl.BlockSpec((BM, D), lambda i: (i, 0)),
            pl.BlockSpec((D, H), lambda i: (0, 0)),
        ],
        out_specs=pl.BlockSpec((BM, H), lambda i: (i, 0)),
        out_shape=jax.ShapeDtypeStruct((N, H), jnp.float32),
    )(x, W0)

    msc0 = pl.pallas_call(
        _scale_body,
        grid=(nblk,),
        in_specs=[
            pl.BlockSpec((BM, 2), lambda i: (i, 0)),
            pl.BlockSpec((BM, H), lambda i: (i, 0)),
        ],
        out_specs=pl.BlockSpec((BM, H), lambda i: (i, 0)),
        out_shape=jax.ShapeDtypeStruct((N, H), jnp.float32),
    )(degt, m0)

    zrows = jnp.zeros((320, H), jnp.float32)
    agg0 = _agg_kernel_build()(src3, dst3, msc0, zrows)

    h1, msc1 = pl.pallas_call(
        _mid_body,
        grid=(nblk,),
        in_specs=[
            pl.BlockSpec((BM, 2), lambda i: (i, 0)),
            pl.BlockSpec((2, BM, 128), lambda i: (0, i, 0)),
            pl.BlockSpec((1, H), lambda i: (0, 0)),
            pl.BlockSpec((H, H), lambda i: (0, 0)),
        ],
        out_specs=[
            pl.BlockSpec((BM, H), lambda i: (i, 0)),
            pl.BlockSpec((BM, H), lambda i: (i, 0)),
        ],
        out_shape=[
            jax.ShapeDtypeStruct((N, H), jnp.float32),
            jax.ShapeDtypeStruct((N, H), jnp.float32),
        ],
    )(degt, agg0, b0r, W1)

    agg1 = _agg_kernel_build()(src3, dst3, msc1, zrows)

    coords = pl.pallas_call(
        _fin_body,
        grid=(nblk,),
        in_specs=[
            pl.BlockSpec((BM, 2), lambda i: (i, 0)),
            pl.BlockSpec((BM, H), lambda i: (i, 0)),
            pl.BlockSpec((2, BM, 128), lambda i: (0, i, 0)),
            pl.BlockSpec((1, H), lambda i: (0, 0)),
            pl.BlockSpec((H, 2), lambda i: (0, 0)),
            pl.BlockSpec((1, 2), lambda i: (0, 0)),
        ],
        out_specs=pl.BlockSpec((BM, 2), lambda i: (i, 0)),
        out_shape=jax.ShapeDtypeStruct((N, 2), jnp.float32),
    )(degt, h1, agg1, b1r, Wout, boutr)

    return coords
